# baseline (speedup 1.0000x reference)
import functools

import jax
import jax.numpy as jnp
from jax import lax
from jax.experimental import pallas as pl
from jax.experimental.pallas import tpu as pltpu
from jax.experimental.pallas import tpu_sc as plsc

_N = 10000
_E = 320000
_D = 128
_G = 64
_C = 40

_K = 128
_NCHUNK = _E // _K
_BLK = 8
_NBLK = (_NCHUNK + _BLK - 1) // _BLK
_BITERS = (_NBLK + 31) // 32
_WCH = 80
_NWCH = _N // _WCH
_WITERS = (_NWCH + 15) // 16


@functools.cache
def _sc_kernels():
  mesh = plsc.VectorSubcoreMesh(core_axis_name="c", subcore_axis_name="s")

  @functools.partial(
      pl.kernel,
      out_type=jax.ShapeDtypeStruct((2, _N, 16), jnp.float32),
      mesh=mesh,
      scratch_types=[
          pltpu.VMEM((_WCH, 16), jnp.float32),
          pltpu.VMEM((_K, 16), jnp.float32),
          pltpu.VMEM((_BLK, _K), jnp.int32),
          pltpu.VMEM_SHARED((_N, 16), jnp.float32),
          pltpu.SemaphoreType.DMA,
      ],
  )
  def sc_count(ei_hbm, out_hbm, zb, ones_v, dblk, cnt_sh, sem):
    c = lax.axis_index("c")
    s = lax.axis_index("s")
    w = c * 16 + s

    @pl.loop(0, _WCH)
    def _(r):
      zb[r, :] = jnp.zeros((16,), jnp.float32)

    @pl.loop(0, _K)
    def _(r):
      ones_v[r, :] = jnp.ones((16,), jnp.float32)

    @pl.loop(0, _WITERS)
    def _(j):
      jj = s + 16 * j

      @pl.when(jj < _NWCH)
      def _():
        pltpu.sync_copy(zb, cnt_sh.at[pl.ds(jj * _WCH, _WCH)])

    plsc.subcore_barrier()

    @pl.loop(0, _BITERS)
    def _(i):
      b = w + 32 * i

      @pl.when(b < _NBLK)
      def _():
        for j in range(_BLK):
          @pl.when(b * _BLK + j < _NCHUNK)
          def _():
            pltpu.sync_copy(ei_hbm.at[1, pl.ds((b * _BLK + j) * _K, _K)],
                            dblk.at[j])
        for j in range(_BLK):
          @pl.when(b * _BLK + j < _NCHUNK)
          def _():
            pltpu.async_copy(ones_v, cnt_sh.at[dblk.at[j]], sem, add=True)
        for j in range(_BLK):
          @pl.when(b * _BLK + j < _NCHUNK)
          def _():
            pltpu.make_async_copy(ones_v, cnt_sh.at[dblk.at[j]], sem).wait()

    plsc.subcore_barrier()

    @pl.loop(0, _WITERS)
    def _(j):
      jj = s + 16 * j

      @pl.when(jj < _NWCH)
      def _():
        off = jj * _WCH
        pltpu.sync_copy(cnt_sh.at[pl.ds(off, _WCH)],
                        out_hbm.at[c].at[pl.ds(off, _WCH)])

  @functools.partial(
      pl.kernel,
      out_type=jax.ShapeDtypeStruct((2, _N, _D), jnp.float32),
      mesh=mesh,
      scratch_types=[
          pltpu.VMEM((_K, _D), jnp.float32),
          pltpu.VMEM((_K, _D), jnp.float32),
          pltpu.VMEM((_BLK, _K), jnp.int32),
          pltpu.VMEM((_BLK, _K), jnp.int32),
          pltpu.VMEM((_BLK, _K), jnp.int32),
          pltpu.VMEM((_BLK, _K), jnp.int32),
          pltpu.VMEM_SHARED((_N, _D), jnp.float32),
          pltpu.SemaphoreType.DMA,
          pltpu.SemaphoreType.DMA,
          pltpu.SemaphoreType.DMA,
          pltpu.SemaphoreType.DMA,
      ],
  )
  def sc_agg(h_hbm, src_hbm, dst_hbm, out_hbm,
             rows0, rows1, sblkA, sblkB, dblkA, dblkB,
             acc_sh, sem0, sem1, semIA, semIB):
    c = lax.axis_index("c")
    s = lax.axis_index("s")

    @pl.loop(0, _K)
    def _(r):
      @pl.loop(0, _D, step=16)
      def _(cc):
        rows0[r, pl.ds(cc, 16)] = jnp.zeros((16,), jnp.float32)

    @pl.loop(0, _WITERS)
    def _(j):
      jj = s + 16 * j

      @pl.when(jj < _NWCH)
      def _():
        pltpu.sync_copy(rows0.at[pl.ds(0, _WCH)],
                        acc_sh.at[pl.ds(jj * _WCH, _WCH)])

    plsc.subcore_barrier()

    w = c * 16 + s
    rbufs = (rows0, rows1)
    rsems = (sem0, sem1)

    def load_idx(b, sblk_b, dblk_b, sem_b):
      @pl.when(b < _NBLK)
      def _():
        pltpu.async_copy(src_hbm.at[pl.ds(b * _BLK, _BLK)], sblk_b, sem_b)
        pltpu.async_copy(dst_hbm.at[pl.ds(b * _BLK, _BLK)], dblk_b, sem_b)

    def wait_idx(b, sblk_b, dblk_b, sem_b):
      @pl.when(b < _NBLK)
      def _():
        pltpu.make_async_copy(
            src_hbm.at[pl.ds(b * _BLK, _BLK)], sblk_b, sem_b).wait()
        pltpu.make_async_copy(
            dst_hbm.at[pl.ds(b * _BLK, _BLK)], dblk_b, sem_b).wait()

    def fire(b, sblk_b, j, rows_b, sem_b):
      @pl.when(b * _BLK + j < _NCHUNK)
      def _():
        pltpu.async_copy(h_hbm.at[sblk_b.at[j]], rows_b, sem_b)

    def drain(b, sblk_b, dblk_b, j, rows_b, sem_b):
      @pl.when(b * _BLK + j < _NCHUNK)
      def _():
        pltpu.make_async_copy(h_hbm.at[sblk_b.at[j]], rows_b, sem_b).wait()
        pltpu.sync_copy(rows_b, acc_sh.at[dblk_b.at[j]], add=True)

    def run_block(b, sblk_b, dblk_b, bn, sblk_n, dblk_n, sem_n):
      @pl.when(b < _NBLK)
      def _():
        for j in range(_BLK):
          if j + 1 < _BLK:
            fire(b, sblk_b, j + 1, rbufs[(j + 1) % 2], rsems[(j + 1) % 2])
          else:
            wait_idx(bn, sblk_n, dblk_n, sem_n)
            fire(bn, sblk_n, 0, rbufs[0], rsems[0])
          drain(b, sblk_b, dblk_b, j, rbufs[j % 2], rsems[j % 2])

    load_idx(w, sblkA, dblkA, semIA)
    wait_idx(w, sblkA, dblkA, semIA)
    fire(w, sblkA, 0, rbufs[0], rsems[0])

    @pl.loop(0, _BITERS, step=2)
    def _(i):
      b = w + 32 * i
      bn = w + 32 * (i + 1)
      bnn = w + 32 * (i + 2)
      load_idx(bn, sblkB, dblkB, semIB)
      run_block(b, sblkA, dblkA, bn, sblkB, dblkB, semIB)
      load_idx(bnn, sblkA, dblkA, semIA)
      run_block(bn, sblkB, dblkB, bnn, sblkA, dblkA, semIA)

    plsc.subcore_barrier()

    @pl.loop(0, _WITERS)
    def _(j):
      jj = s + 16 * j

      @pl.when(jj < _NWCH)
      def _():
        off = jj * _WCH
        pltpu.sync_copy(acc_sh.at[pl.ds(off, _WCH)],
                        out_hbm.at[c].at[pl.ds(off, _WCH)])

  return sc_count, sc_agg


_R = 1000
_NB = _N // _R


def _tc1a_body(x_ref, w1_ref, h1_ref):
  h1_ref[...] = jnp.dot(x_ref[...], w1_ref[...],
                        preferred_element_type=jnp.float32)


def _tc1a(x, w1):
  return pl.pallas_call(
      _tc1a_body,
      grid=(_NB,),
      in_specs=[
          pl.BlockSpec((_R, _D), lambda i: (i, 0)),
          pl.BlockSpec((_D, _D), lambda i: (0, 0)),
      ],
      out_specs=pl.BlockSpec((_R, _D), lambda i: (i, 0)),
      out_shape=jax.ShapeDtypeStruct((_N, _D), jnp.float32),
  )(x, w1)


def _tc1b_body(cnt_ref, h1_ref, h1p_ref, dinv_ref):
  cnt = cnt_ref[0, :, 0:1] + cnt_ref[1, :, 0:1]
  dinv = lax.rsqrt(cnt + 1.0)
  h1p_ref[...] = h1_ref[...] * dinv
  dinv_ref[...] = dinv


def _tc1b(cnt, h1):
  return pl.pallas_call(
      _tc1b_body,
      grid=(_NB,),
      in_specs=[
          pl.BlockSpec((2, _R, 16), lambda i: (0, i, 0)),
          pl.BlockSpec((_R, _D), lambda i: (i, 0)),
      ],
      out_specs=[
          pl.BlockSpec((_R, _D), lambda i: (i, 0)),
          pl.BlockSpec((_R, 1), lambda i: (i, 0)),
      ],
      out_shape=[
          jax.ShapeDtypeStruct((_N, _D), jnp.float32),
          jax.ShapeDtypeStruct((_N, 1), jnp.float32),
      ],
  )(cnt, h1)


def _tc2_body(a_ref, h1p_ref, dinv_ref, b1_ref, w2_ref, h2p_ref):
  dinv = dinv_ref[...]
  z = dinv * (a_ref[0] + a_ref[1] + h1p_ref[...]) + b1_ref[...]
  sz = z * jax.nn.sigmoid(z)
  h2p_ref[...] = jnp.dot(sz, w2_ref[...],
                         preferred_element_type=jnp.float32) * dinv


def _tc2(acc, h1p, dinv, b1, w2):
  return pl.pallas_call(
      _tc2_body,
      grid=(_NB,),
      in_specs=[
          pl.BlockSpec((2, _R, _D), lambda i: (0, i, 0)),
          pl.BlockSpec((_R, _D), lambda i: (i, 0)),
          pl.BlockSpec((_R, 1), lambda i: (i, 0)),
          pl.BlockSpec((1, _D), lambda i: (0, 0)),
          pl.BlockSpec((_D, _D), lambda i: (0, 0)),
      ],
      out_specs=pl.BlockSpec((_R, _D), lambda i: (i, 0)),
      out_shape=jax.ShapeDtypeStruct((_N, _D), jnp.float32),
  )(acc, h1p, dinv, b1, w2)


def _tc3_body(a_ref, h2p_ref, dinv_ref, b2_ref, batch_ref, wr_ref, br_ref,
              out_ref, sums, counts):
  i = pl.program_id(0)

  @pl.when(i == 0)
  def _():
    sums[...] = jnp.zeros_like(sums)
    counts[...] = jnp.zeros_like(counts)

  dinv = dinv_ref[...]
  z = dinv * (a_ref[0] + a_ref[1] + h2p_ref[...]) + b2_ref[...]
  sz = z * jax.nn.sigmoid(z)
  b = batch_ref[0]
  g = lax.broadcasted_iota(jnp.int32, (_G, _R), 0)
  onehot = (g == b).astype(jnp.float32)
  sums[...] += jnp.dot(onehot, sz, preferred_element_type=jnp.float32)
  counts[...] += jnp.sum(onehot, axis=1, keepdims=True)

  @pl.when(i == _NB - 1)
  def _():
    pooled = sums[...] / jnp.maximum(counts[...], 1.0)
    logits = jnp.maximum(
        jnp.dot(pooled, wr_ref[...], preferred_element_type=jnp.float32)
        + br_ref[...], 0.0)
    m = jnp.max(logits, axis=1, keepdims=True)
    lse = m + jnp.log(jnp.sum(jnp.exp(logits - m), axis=1, keepdims=True))
    out_ref[...] = logits - lse


def _tc3(acc, h2p, dinv, b2, batch3, wr, br):
  return pl.pallas_call(
      _tc3_body,
      grid=(_NB,),
      in_specs=[
          pl.BlockSpec((2, _R, _D), lambda i: (0, i, 0)),
          pl.BlockSpec((_R, _D), lambda i: (i, 0)),
          pl.BlockSpec((_R, 1), lambda i: (i, 0)),
          pl.BlockSpec((1, _D), lambda i: (0, 0)),
          pl.BlockSpec((1, 1, _R), lambda i: (i, 0, 0)),
          pl.BlockSpec((_D, _C), lambda i: (0, 0)),
          pl.BlockSpec((1, _C), lambda i: (0, 0)),
      ],
      out_specs=pl.BlockSpec((_G, _C), lambda i: (0, 0)),
      out_shape=jax.ShapeDtypeStruct((_G, _C), jnp.float32),
      scratch_shapes=[
          pltpu.VMEM((_G, _D), jnp.float32),
          pltpu.VMEM((_G, 1), jnp.float32),
      ],
  )(acc, h2p, dinv, b2, batch3, wr, br)


def kernel(x, edge_index, batch, W1, b1, W2, b2, Wr, br):
  sc_count, sc_agg = _sc_kernels()
  cnt = sc_count(edge_index)
  src = edge_index[0].reshape(_NCHUNK, _K)
  dst = edge_index[1].reshape(_NCHUNK, _K)
  h1 = _tc1a(x, W1)
  h1p, dinv = _tc1b(cnt, h1)
  acc1 = sc_agg(h1p, src, dst)
  h2p = _tc2(acc1, h1p, dinv, b1.reshape(1, _D), W2)
  acc2 = sc_agg(h2p, src, dst)
  return _tc3(acc2, h2p, dinv, b2.reshape(1, _D),
              batch.reshape(_NB, 1, _R), Wr, br.reshape(1, _C))


# speedup vs baseline: 1.1039x; 1.1039x over previous
import functools

import jax
import jax.numpy as jnp
from jax import lax
from jax.experimental import pallas as pl
from jax.experimental.pallas import tpu as pltpu
from jax.experimental.pallas import tpu_sc as plsc

_N = 10000
_E = 320000
_D = 128
_G = 64
_C = 40

_K = 128
_NCHUNK = _E // _K
_BLK = 8
_NBLK = (_NCHUNK + _BLK - 1) // _BLK
_BITERS = (_NBLK + 31) // 32
_WCH = 80
_NWCH = _N // _WCH
_WITERS = (_NWCH + 15) // 16


@functools.cache
def _sc_kernels():
  mesh = plsc.VectorSubcoreMesh(core_axis_name="c", subcore_axis_name="s")

  @functools.partial(
      pl.kernel,
      out_type=jax.ShapeDtypeStruct((2, _N, 16), jnp.float32),
      mesh=mesh,
      scratch_types=[
          pltpu.VMEM((_WCH, 16), jnp.float32),
          pltpu.VMEM((_K, 16), jnp.float32),
          pltpu.VMEM((_BLK, _K), jnp.int32),
          pltpu.VMEM_SHARED((_N, 16), jnp.float32),
          pltpu.SemaphoreType.DMA,
      ],
  )
  def sc_count(ei_hbm, out_hbm, zb, ones_v, dblk, cnt_sh, sem):
    c = lax.axis_index("c")
    s = lax.axis_index("s")
    w = c * 16 + s

    @pl.loop(0, _WCH)
    def _(r):
      zb[r, :] = jnp.zeros((16,), jnp.float32)

    @pl.loop(0, _K)
    def _(r):
      ones_v[r, :] = jnp.ones((16,), jnp.float32)

    @pl.loop(0, _WITERS)
    def _(j):
      jj = s + 16 * j

      @pl.when(jj < _NWCH)
      def _():
        pltpu.sync_copy(zb, cnt_sh.at[pl.ds(jj * _WCH, _WCH)])

    plsc.subcore_barrier()

    @pl.loop(0, _BITERS)
    def _(i):
      b = w + 32 * i

      @pl.when(b < _NBLK)
      def _():
        for j in range(_BLK):
          @pl.when(b * _BLK + j < _NCHUNK)
          def _():
            pltpu.async_copy(ei_hbm.at[1, pl.ds((b * _BLK + j) * _K, _K)],
                             dblk.at[j], sem)
        for j in range(_BLK):
          @pl.when(b * _BLK + j < _NCHUNK)
          def _():
            pltpu.make_async_copy(
                ei_hbm.at[1, pl.ds((b * _BLK + j) * _K, _K)],
                dblk.at[j], sem).wait()
        for j in range(_BLK):
          @pl.when(b * _BLK + j < _NCHUNK)
          def _():
            pltpu.async_copy(ones_v, cnt_sh.at[dblk.at[j]], sem, add=True)
        for j in range(_BLK):
          @pl.when(b * _BLK + j < _NCHUNK)
          def _():
            pltpu.make_async_copy(ones_v, cnt_sh.at[dblk.at[j]], sem).wait()

    plsc.subcore_barrier()

    @pl.loop(0, _WITERS)
    def _(j):
      jj = s + 16 * j

      @pl.when(jj < _NWCH)
      def _():
        off = jj * _WCH
        pltpu.sync_copy(cnt_sh.at[pl.ds(off, _WCH)],
                        out_hbm.at[c].at[pl.ds(off, _WCH)])

  @functools.partial(
      pl.kernel,
      out_type=jax.ShapeDtypeStruct((2, _N, _D), jnp.float32),
      mesh=mesh,
      scratch_types=[
          pltpu.VMEM((_K, _D), jnp.float32),
          pltpu.VMEM((_K, _D), jnp.float32),
          pltpu.VMEM((_BLK, _K), jnp.int32),
          pltpu.VMEM((_BLK, _K), jnp.int32),
          pltpu.VMEM((_BLK, _K), jnp.int32),
          pltpu.VMEM((_BLK, _K), jnp.int32),
          pltpu.VMEM_SHARED((_N, _D), jnp.float32),
          pltpu.SemaphoreType.DMA,
          pltpu.SemaphoreType.DMA,
          pltpu.SemaphoreType.DMA,
          pltpu.SemaphoreType.DMA,
      ],
  )
  def sc_agg(h_hbm, src_hbm, dst_hbm, out_hbm,
             rows0, rows1, sblkA, sblkB, dblkA, dblkB,
             acc_sh, sem0, sem1, semIA, semIB):
    c = lax.axis_index("c")
    s = lax.axis_index("s")

    @pl.loop(0, _K)
    def _(r):
      @pl.loop(0, _D, step=16)
      def _(cc):
        rows0[r, pl.ds(cc, 16)] = jnp.zeros((16,), jnp.float32)

    @pl.loop(0, _WITERS)
    def _(j):
      jj = s + 16 * j

      @pl.when(jj < _NWCH)
      def _():
        pltpu.sync_copy(rows0.at[pl.ds(0, _WCH)],
                        acc_sh.at[pl.ds(jj * _WCH, _WCH)])

    plsc.subcore_barrier()

    w = c * 16 + s
    rbufs = (rows0, rows1)
    rsems = (sem0, sem1)

    def load_idx(b, sblk_b, dblk_b, sem_b):
      @pl.when(b < _NBLK)
      def _():
        pltpu.async_copy(src_hbm.at[pl.ds(b * _BLK, _BLK)], sblk_b, sem_b)
        pltpu.async_copy(dst_hbm.at[pl.ds(b * _BLK, _BLK)], dblk_b, sem_b)

    def wait_idx(b, sblk_b, dblk_b, sem_b):
      @pl.when(b < _NBLK)
      def _():
        pltpu.make_async_copy(
            src_hbm.at[pl.ds(b * _BLK, _BLK)], sblk_b, sem_b).wait()
        pltpu.make_async_copy(
            dst_hbm.at[pl.ds(b * _BLK, _BLK)], dblk_b, sem_b).wait()

    def fire(b, sblk_b, j, rows_b, sem_b):
      @pl.when(b * _BLK + j < _NCHUNK)
      def _():
        pltpu.async_copy(h_hbm.at[sblk_b.at[j]], rows_b, sem_b)

    def drain(b, sblk_b, dblk_b, j, rows_b, sem_b):
      @pl.when(b * _BLK + j < _NCHUNK)
      def _():
        pltpu.make_async_copy(h_hbm.at[sblk_b.at[j]], rows_b, sem_b).wait()
        pltpu.sync_copy(rows_b, acc_sh.at[dblk_b.at[j]], add=True)

    def run_block(b, sblk_b, dblk_b, bn, sblk_n, dblk_n, sem_n):
      @pl.when(b < _NBLK)
      def _():
        for j in range(_BLK):
          if j + 1 < _BLK:
            fire(b, sblk_b, j + 1, rbufs[(j + 1) % 2], rsems[(j + 1) % 2])
          else:
            wait_idx(bn, sblk_n, dblk_n, sem_n)
            fire(bn, sblk_n, 0, rbufs[0], rsems[0])
          drain(b, sblk_b, dblk_b, j, rbufs[j % 2], rsems[j % 2])

    load_idx(w, sblkA, dblkA, semIA)
    wait_idx(w, sblkA, dblkA, semIA)
    fire(w, sblkA, 0, rbufs[0], rsems[0])

    @pl.loop(0, _BITERS, step=2)
    def _(i):
      b = w + 32 * i
      bn = w + 32 * (i + 1)
      bnn = w + 32 * (i + 2)
      load_idx(bn, sblkB, dblkB, semIB)
      run_block(b, sblkA, dblkA, bn, sblkB, dblkB, semIB)
      load_idx(bnn, sblkA, dblkA, semIA)
      run_block(bn, sblkB, dblkB, bnn, sblkA, dblkA, semIA)

    plsc.subcore_barrier()

    @pl.loop(0, _WITERS)
    def _(j):
      jj = s + 16 * j

      @pl.when(jj < _NWCH)
      def _():
        off = jj * _WCH
        pltpu.sync_copy(acc_sh.at[pl.ds(off, _WCH)],
                        out_hbm.at[c].at[pl.ds(off, _WCH)])

  return sc_count, sc_agg


_R = 1000
_NB = _N // _R


def _tc1a_body(x_ref, w1_ref, h1_ref):
  h1_ref[...] = jnp.dot(x_ref[...], w1_ref[...],
                        preferred_element_type=jnp.float32)


def _tc1a(x, w1):
  return pl.pallas_call(
      _tc1a_body,
      grid=(_NB,),
      in_specs=[
          pl.BlockSpec((_R, _D), lambda i: (i, 0)),
          pl.BlockSpec((_D, _D), lambda i: (0, 0)),
      ],
      out_specs=pl.BlockSpec((_R, _D), lambda i: (i, 0)),
      out_shape=jax.ShapeDtypeStruct((_N, _D), jnp.float32),
  )(x, w1)


def _tc1b_body(cnt_ref, h1_ref, h1p_ref, dinv_ref):
  cnt = cnt_ref[0, :, 0:1] + cnt_ref[1, :, 0:1]
  dinv = lax.rsqrt(cnt + 1.0)
  h1p_ref[...] = h1_ref[...] * dinv
  dinv_ref[...] = dinv


def _tc1b(cnt, h1):
  return pl.pallas_call(
      _tc1b_body,
      grid=(_NB,),
      in_specs=[
          pl.BlockSpec((2, _R, 16), lambda i: (0, i, 0)),
          pl.BlockSpec((_R, _D), lambda i: (i, 0)),
      ],
      out_specs=[
          pl.BlockSpec((_R, _D), lambda i: (i, 0)),
          pl.BlockSpec((_R, 1), lambda i: (i, 0)),
      ],
      out_shape=[
          jax.ShapeDtypeStruct((_N, _D), jnp.float32),
          jax.ShapeDtypeStruct((_N, 1), jnp.float32),
      ],
  )(cnt, h1)


def _tc2_body(a_ref, h1p_ref, dinv_ref, b1_ref, w2_ref, h2p_ref):
  dinv = dinv_ref[...]
  z = dinv * (a_ref[0] + a_ref[1] + h1p_ref[...]) + b1_ref[...]
  sz = z * jax.nn.sigmoid(z)
  h2p_ref[...] = jnp.dot(sz, w2_ref[...],
                         preferred_element_type=jnp.float32) * dinv


def _tc2(acc, h1p, dinv, b1, w2):
  return pl.pallas_call(
      _tc2_body,
      grid=(_NB,),
      in_specs=[
          pl.BlockSpec((2, _R, _D), lambda i: (0, i, 0)),
          pl.BlockSpec((_R, _D), lambda i: (i, 0)),
          pl.BlockSpec((_R, 1), lambda i: (i, 0)),
          pl.BlockSpec((1, _D), lambda i: (0, 0)),
          pl.BlockSpec((_D, _D), lambda i: (0, 0)),
      ],
      out_specs=pl.BlockSpec((_R, _D), lambda i: (i, 0)),
      out_shape=jax.ShapeDtypeStruct((_N, _D), jnp.float32),
  )(acc, h1p, dinv, b1, w2)


def _tc3_body(a_ref, h2p_ref, dinv_ref, b2_ref, batch_ref, wr_ref, br_ref,
              out_ref, sums, counts):
  i = pl.program_id(0)

  @pl.when(i == 0)
  def _():
    sums[...] = jnp.zeros_like(sums)
    counts[...] = jnp.zeros_like(counts)

  dinv = dinv_ref[...]
  z = dinv * (a_ref[0] + a_ref[1] + h2p_ref[...]) + b2_ref[...]
  sz = z * jax.nn.sigmoid(z)
  b = batch_ref[0]
  g = lax.broadcasted_iota(jnp.int32, (_G, _R), 0)
  onehot = (g == b).astype(jnp.float32)
  sums[...] += jnp.dot(onehot, sz, preferred_element_type=jnp.float32)
  counts[...] += jnp.sum(onehot, axis=1, keepdims=True)

  @pl.when(i == _NB - 1)
  def _():
    pooled = sums[...] / jnp.maximum(counts[...], 1.0)
    logits = jnp.maximum(
        jnp.dot(pooled, wr_ref[...], preferred_element_type=jnp.float32)
        + br_ref[...], 0.0)
    m = jnp.max(logits, axis=1, keepdims=True)
    lse = m + jnp.log(jnp.sum(jnp.exp(logits - m), axis=1, keepdims=True))
    out_ref[...] = logits - lse


def _tc3(acc, h2p, dinv, b2, batch3, wr, br):
  return pl.pallas_call(
      _tc3_body,
      grid=(_NB,),
      in_specs=[
          pl.BlockSpec((2, _R, _D), lambda i: (0, i, 0)),
          pl.BlockSpec((_R, _D), lambda i: (i, 0)),
          pl.BlockSpec((_R, 1), lambda i: (i, 0)),
          pl.BlockSpec((1, _D), lambda i: (0, 0)),
          pl.BlockSpec((1, 1, _R), lambda i: (i, 0, 0)),
          pl.BlockSpec((_D, _C), lambda i: (0, 0)),
          pl.BlockSpec((1, _C), lambda i: (0, 0)),
      ],
      out_specs=pl.BlockSpec((_G, _C), lambda i: (0, 0)),
      out_shape=jax.ShapeDtypeStruct((_G, _C), jnp.float32),
      scratch_shapes=[
          pltpu.VMEM((_G, _D), jnp.float32),
          pltpu.VMEM((_G, 1), jnp.float32),
      ],
  )(acc, h2p, dinv, b2, batch3, wr, br)


def kernel(x, edge_index, batch, W1, b1, W2, b2, Wr, br):
  sc_count, sc_agg = _sc_kernels()
  cnt = sc_count(edge_index)
  src = edge_index[0].reshape(_NCHUNK, _K)
  dst = edge_index[1].reshape(_NCHUNK, _K)
  h1 = _tc1a(x, W1)
  h1p, dinv = _tc1b(cnt, h1)
  acc1 = sc_agg(h1p, src, dst)
  h2p = _tc2(acc1, h1p, dinv, b1.reshape(1, _D), W2)
  acc2 = sc_agg(h2p, src, dst)
  return _tc3(acc2, h2p, dinv, b2.reshape(1, _D),
              batch.reshape(_NB, 1, _R), Wr, br.reshape(1, _C))
